# baseline (speedup 1.0000x reference)
import jax
import jax.numpy as jnp
from jax.experimental import pallas as pl
from jax.experimental.pallas import tpu as pltpu

_B, _L, _CL, _D, _E, _G = 32, 64, 512, 1024, 8, 2
_F = 3 * _CL
_R = _B * _L
_NE = _E + _G
_EPS = 1e-9

_DB = 512
_ND = _D // _DB
_NS = _E + 1


def _moe_kernel(logits_ref, mask_ref, flat_ref, we_ref, wg_ref, b_ref,
                out_ref, fbf_ref, grow_ref, acc_ref):
    d = pl.program_id(0)
    e = pl.program_id(1)

    @pl.when((d == 0) & (e == 0))
    def _once():
        fbf_ref[...] = flat_ref[...].astype(jnp.bfloat16)
        logits = logits_ref[...]
        maskf = jnp.where(mask_ref[...] == 1, 1.0, 0.0).astype(jnp.float32)
        g = jax.nn.softmax(logits, axis=1) * maskf
        g = g / (jnp.sum(g, axis=1, keepdims=True) + _EPS)
        grow = jnp.repeat(g, _L, axis=0)
        grow_ref[...] = jnp.concatenate(
            [grow, jnp.ones((_R, _G), jnp.float32),
             jnp.zeros((_R, 16 - _NE), jnp.float32)], axis=1)

    @pl.when(e < _E)
    def _expert():
        y = jnp.dot(fbf_ref[...], we_ref[0].astype(jnp.bfloat16),
                    preferred_element_type=jnp.float32)
        lane = jax.lax.broadcasted_iota(jnp.int32, (_R, _E), 1)
        scale = jnp.sum(jnp.where(lane == e, grow_ref[:, :_E], 0.0),
                        axis=1, keepdims=True)
        contrib = scale * y

        @pl.when(e == 0)
        def _init():
            bias = jnp.dot(grow_ref[...], b_ref[...],
                           preferred_element_type=jnp.float32)
            acc_ref[...] = (bias + contrib).astype(jnp.bfloat16)

        @pl.when(e != 0)
        def _accum():
            acc_ref[...] += contrib.astype(jnp.bfloat16)

    @pl.when(e == _E)
    def _general():
        wsum = (wg_ref[0] + wg_ref[1]).astype(jnp.bfloat16)
        y = jnp.dot(fbf_ref[...], wsum, preferred_element_type=jnp.float32)
        out_ref[...] = acc_ref[...].astype(jnp.float32) + y


def kernel(cycle_curve_data, logits, moe_masks, We, be, Wg, bg):
    flat = cycle_curve_data.reshape(_R, _F)
    b_all = jnp.zeros((16, _D), jnp.float32)
    b_all = b_all.at[:_E].set(be).at[_E:_NE].set(bg)

    out = pl.pallas_call(
        _moe_kernel,
        grid=(_ND, _NS),
        in_specs=[
            pl.BlockSpec((_B, _E), lambda d, e: (0, 0)),
            pl.BlockSpec((_B, _E), lambda d, e: (0, 0)),
            pl.BlockSpec((_R, _F), lambda d, e: (0, 0)),
            pl.BlockSpec((1, _F, _DB),
                         lambda d, e: (jnp.minimum(e, _E - 1), 0, d)),
            pl.BlockSpec((_G, _F, _DB), lambda d, e: (0, 0, d)),
            pl.BlockSpec((16, _DB), lambda d, e: (0, d)),
        ],
        out_specs=pl.BlockSpec((_R, _DB), lambda d, e: (0, d)),
        out_shape=jax.ShapeDtypeStruct((_R, _D), jnp.float32),
        scratch_shapes=[
            pltpu.VMEM((_R, _F), jnp.bfloat16),
            pltpu.VMEM((_R, 16), jnp.float32),
            pltpu.VMEM((_R, _DB), jnp.bfloat16),
        ],
    )(logits, moe_masks.astype(jnp.int32), flat, We, Wg, b_all)

    final_out = out.reshape(_B, _L, _D)
    aug_loss = jnp.zeros((), dtype=jnp.float32)
    guide_loss = jnp.zeros((), dtype=jnp.float32)
    return (final_out, aug_loss, guide_loss)


# speedup vs baseline: 1.0180x; 1.0180x over previous
import jax
import jax.numpy as jnp
from jax.experimental import pallas as pl
from jax.experimental.pallas import tpu as pltpu

_B, _L, _CL, _D, _E, _G = 32, 64, 512, 1024, 8, 2
_F = 3 * _CL
_R = _B * _L
_NE = _E + _G
_EPS = 1e-9

_DB = 512
_ND = _D // _DB
_NS = _E + 1


def _moe_kernel(logits_ref, mask_ref, flat_ref, we_ref, wg_ref, b_ref,
                out_ref, fbf_ref, grow_ref):
    d = pl.program_id(0)
    e = pl.program_id(1)

    @pl.when((d == 0) & (e == 0))
    def _once():
        fbf_ref[...] = flat_ref[...].astype(jnp.bfloat16)
        logits = logits_ref[...]
        maskf = jnp.where(mask_ref[...] == 1, 1.0, 0.0).astype(jnp.float32)
        g = jax.nn.softmax(logits, axis=1) * maskf
        g = g / (jnp.sum(g, axis=1, keepdims=True) + _EPS)
        grow = jnp.repeat(g, _L, axis=0)
        grow_ref[...] = jnp.concatenate(
            [grow, jnp.ones((_R, _G), jnp.float32),
             jnp.zeros((_R, 16 - _NE), jnp.float32)], axis=1)

    @pl.when(e == 0)
    def _bias_init():
        out_ref[...] = jnp.dot(grow_ref[...], b_ref[...],
                               preferred_element_type=jnp.float32)

    @pl.when(e < _E)
    def _expert():
        y = jnp.dot(fbf_ref[...], we_ref[0].astype(jnp.bfloat16),
                    preferred_element_type=jnp.float32)
        lane = jax.lax.broadcasted_iota(jnp.int32, (_R, _E), 1)
        scale = jnp.sum(jnp.where(lane == e, grow_ref[:, :_E], 0.0),
                        axis=1, keepdims=True)
        out_ref[...] += scale * y

    @pl.when(e == _E)
    def _general():
        wsum = (wg_ref[0] + wg_ref[1]).astype(jnp.bfloat16)
        y = jnp.dot(fbf_ref[...], wsum, preferred_element_type=jnp.float32)
        rounded = out_ref[...].astype(jnp.bfloat16).astype(jnp.float32)
        out_ref[...] = rounded + y


def kernel(cycle_curve_data, logits, moe_masks, We, be, Wg, bg):
    flat = cycle_curve_data.reshape(_R, _F)
    b_all = jnp.zeros((16, _D), jnp.float32)
    b_all = b_all.at[:_E].set(be).at[_E:_NE].set(bg)

    out = pl.pallas_call(
        _moe_kernel,
        grid=(_ND, _NS),
        in_specs=[
            pl.BlockSpec((_B, _E), lambda d, e: (0, 0)),
            pl.BlockSpec((_B, _E), lambda d, e: (0, 0)),
            pl.BlockSpec((_R, _F), lambda d, e: (0, 0)),
            pl.BlockSpec((1, _F, _DB),
                         lambda d, e: (jnp.minimum(e, _E - 1), 0, d)),
            pl.BlockSpec((_G, _F, _DB), lambda d, e: (0, 0, d)),
            pl.BlockSpec((16, _DB), lambda d, e: (0, d)),
        ],
        out_specs=pl.BlockSpec((_R, _DB), lambda d, e: (0, d)),
        out_shape=jax.ShapeDtypeStruct((_R, _D), jnp.float32),
        scratch_shapes=[
            pltpu.VMEM((_R, _F), jnp.bfloat16),
            pltpu.VMEM((_R, 16), jnp.float32),
        ],
    )(logits, moe_masks.astype(jnp.int32), flat, We, Wg, b_all)

    final_out = out.reshape(_B, _L, _D)
    aug_loss = jnp.zeros((), dtype=jnp.float32)
    guide_loss = jnp.zeros((), dtype=jnp.float32)
    return (final_out, aug_loss, guide_loss)
